# baseline (speedup 1.0000x reference)
import functools

import numpy as np
import jax
import jax.numpy as jnp
from jax import lax
from jax.experimental import pallas as pl
from jax.experimental.pallas import tpu as pltpu
from jax.experimental.pallas import tpu_sc as plsc

_FREQ_MASKS = 2
_TIME_MASKS = 10
_FREQ_WIDTH = 27
_TIME_WIDTH = 0.05

_CH = 256
_NW = 32


def _trace_time_constants(B, F):
    with jax.ensure_compile_time_eval():
        key = jax.random.key(42)
        kf1, kf2, kt1, kt2 = jax.random.split(key, 4)
        x_left = jax.random.randint(kf1, (B, _FREQ_MASKS), 0, F - _FREQ_WIDTH + 1)
        wf = jax.random.randint(kf2, (B, _FREQ_MASKS), 0, _FREQ_WIDTH + 1)
        f_idx = jnp.arange(F)
        fmask = ((f_idx[None, None, :] >= x_left[:, :, None])
                 & (f_idx[None, None, :] < (x_left + wf)[:, :, None])).any(axis=1)
        keep_f = jnp.logical_not(fmask).astype(jnp.float32)
        u1 = jax.random.uniform(kt1, (B, _TIME_MASKS))
        u2 = jax.random.uniform(kt2, (B, _TIME_MASKS))
        u1p = jnp.zeros((B, 128), jnp.float32).at[:, :_TIME_MASKS].set(u1)
        u2p = jnp.zeros((B, 128), jnp.float32).at[:, :_TIME_MASKS].set(u2)
    return np.asarray(keep_f), np.asarray(u1p), np.asarray(u2p)


def _sc_body(x_hbm, ys_hbm, es_hbm, kf_hbm, out_hbm,
             buf, kf_v, ys_v, es_v, *, B, T, F):
    bpw = B // _NW
    nch = T // _CH
    wid = lax.axis_index("s") * 2 + lax.axis_index("c")
    lane = lax.iota(jnp.int32, 16)
    zv = jnp.zeros((16,), jnp.float32)
    for k in range(bpw):
        b = wid * bpw + k
        pltpu.sync_copy(kf_hbm.at[b], kf_v)
        pltpu.sync_copy(ys_hbm.at[b, pl.ds(0, 16)], ys_v)
        pltpu.sync_copy(es_hbm.at[b, pl.ds(0, 16)], es_v)
        kfr = [kf_v[pl.ds(16 * v, 16)] for v in range(8)]
        ysvec = ys_v[...]
        esvec = es_v[...]
        ys = [ysvec[m] for m in range(_TIME_MASKS)]
        es = [esvec[m] for m in range(_TIME_MASKS)]

        def chunk_step(c, _):
            c0 = c * _CH
            pltpu.sync_copy(x_hbm.at[b, pl.ds(c0, _CH)], buf)

            def row_mul(r, _):
                for v in range(8):
                    sl = pl.ds(16 * v, 16)
                    buf[r, sl] = buf[r, sl] * kfr[v]
                return 0

            lax.fori_loop(0, _CH, row_mul, 0, unroll=2)
            for m in range(_TIME_MASKS):
                lo = jnp.clip(ys[m] - c0, 0, _CH)
                hi = jnp.clip(es[m] - c0, 0, _CH)

                def row_zero(r, _):
                    for v in range(8):
                        buf[r, pl.ds(16 * v, 16)] = zv
                    return 0

                lax.fori_loop(lo, hi, row_zero, 0)
            pltpu.sync_copy(buf, out_hbm.at[b, pl.ds(c0, _CH)])
            return 0

        lax.fori_loop(0, nch, chunk_step, 0)


def kernel(input_spec, length):
    B, T, F = input_spec.shape
    keep_f, u1p, u2p = _trace_time_constants(B, F)
    u1 = u1p[:, :_TIME_MASKS]
    u2 = u2p[:, :_TIME_MASKS]
    len32 = length.astype(jnp.int32)
    tw = jnp.maximum(1, (len32.astype(jnp.float32) * _TIME_WIDTH).astype(jnp.int32))
    y_max = jnp.maximum(1, len32 - tw)
    y_left = jnp.minimum(jnp.floor(u1 * (y_max[:, None] + 1).astype(jnp.float32)).astype(jnp.int32), y_max[:, None])
    wt = jnp.minimum(jnp.floor(u2 * (tw[:, None] + 1).astype(jnp.float32)).astype(jnp.int32), tw[:, None])
    pad = jnp.zeros((B, 128), jnp.int32)
    ys_pad = pad.at[:, :_TIME_MASKS].set(y_left)
    es_pad = pad.at[:, :_TIME_MASKS].set(y_left + wt)

    mesh = plsc.VectorSubcoreMesh(core_axis_name="c", subcore_axis_name="s")
    sc_fn = functools.partial(
        pl.kernel,
        mesh=mesh,
        out_type=jax.ShapeDtypeStruct((B, T, F), jnp.float32),
        scratch_types=[
            pltpu.VMEM((_CH, F), jnp.float32),
            pltpu.VMEM((F,), jnp.float32),
            pltpu.VMEM((16,), jnp.int32),
            pltpu.VMEM((16,), jnp.int32),
        ],
    )(functools.partial(_sc_body, B=B, T=T, F=F))
    out = sc_fn(input_spec, ys_pad, es_pad, jnp.asarray(keep_f))
    return (out, length)


# speedup vs baseline: 2.0871x; 2.0871x over previous
import functools

import numpy as np
import jax
import jax.numpy as jnp
from jax.experimental import pallas as pl
from jax.experimental.pallas import tpu as pltpu

_FREQ_MASKS = 2
_TIME_MASKS = 10
_FREQ_WIDTH = 27
_TIME_WIDTH = 0.05
_WIN = 256


def _trace_time_constants(B, F):
    with jax.ensure_compile_time_eval():
        key = jax.random.key(42)
        kf1, kf2, kt1, kt2 = jax.random.split(key, 4)
        x_left = jax.random.randint(kf1, (B, _FREQ_MASKS), 0, F - _FREQ_WIDTH + 1)
        wf = jax.random.randint(kf2, (B, _FREQ_MASKS), 0, _FREQ_WIDTH + 1)
        f_idx = jnp.arange(F)
        fmask = ((f_idx[None, None, :] >= x_left[:, :, None])
                 & (f_idx[None, None, :] < (x_left + wf)[:, :, None])).any(axis=1)
        keep_f = jnp.logical_not(fmask).astype(jnp.float32)
        u1 = jax.random.uniform(kt1, (B, _TIME_MASKS))
        u2 = jax.random.uniform(kt2, (B, _TIME_MASKS))
    return (np.asarray(keep_f).reshape(B, 1, F),
            np.asarray(u1), np.asarray(u2))


def _mask_kernel(len_ref, u1_ref, u2_ref, kf_ref, x_ref, o_ref, *, t_blk, bb):
    i = pl.program_id(0)
    o_ref[...] = x_ref[...] * kf_ref[...]
    for k in range(bb):
        b = i * bb + k
        lenb = len_ref[b]
        len_f = lenb.astype(jnp.float32)
        tw = jnp.maximum(1, (len_f * _TIME_WIDTH).astype(jnp.int32))
        y_max = jnp.maximum(1, lenb - tw)
        ymf = (y_max + 1).astype(jnp.float32)
        twf = (tw + 1).astype(jnp.float32)
        for m in range(_TIME_MASKS):
            u1 = u1_ref[b, m]
            u2 = u2_ref[b, m]
            y = jnp.minimum(jnp.floor(u1 * ymf).astype(jnp.int32), y_max)
            w = jnp.minimum(jnp.floor(u2 * twf).astype(jnp.int32), tw)
            s = jnp.minimum((y // 8) * 8, t_blk - _WIN)
            ti = jax.lax.broadcasted_iota(jnp.int32, (_WIN, 1), 0) + s
            keepm = jnp.where((ti >= y) & (ti < y + w), 0.0, 1.0)
            o_ref[k, pl.ds(s, _WIN), :] = o_ref[k, pl.ds(s, _WIN), :] * keepm


def kernel(input_spec, length):
    B, T, F = input_spec.shape
    keep_f, u1, u2 = _trace_time_constants(B, F)
    len32 = length.astype(jnp.int32)

    T_BLK = T
    BB = 4
    grid = (B // BB,)
    out = pl.pallas_call(
        functools.partial(_mask_kernel, t_blk=T_BLK, bb=BB),
        grid_spec=pltpu.PrefetchScalarGridSpec(
            num_scalar_prefetch=3,
            grid=grid,
            in_specs=[
                pl.BlockSpec((BB, 1, F), lambda i, *_: (i, 0, 0)),
                pl.BlockSpec((BB, T_BLK, F), lambda i, *_: (i, 0, 0)),
            ],
            out_specs=pl.BlockSpec((BB, T_BLK, F), lambda i, *_: (i, 0, 0)),
        ),
        out_shape=jax.ShapeDtypeStruct((B, T, F), input_spec.dtype),
    )(len32, jnp.asarray(u1), jnp.asarray(u2), jnp.asarray(keep_f), input_spec)
    return (out, length)
